# baseline (speedup 1.0000x reference)
import functools

import numpy as np
import jax
import jax.numpy as jnp
from jax import lax
from jax.experimental import pallas as pl
from jax.experimental.pallas import tpu as pltpu
from jax.experimental.pallas import tpu_sc as plsc

N = 10000
E = 320000
H = 128
DB = 16
G = 256
NL = 3

NS = 16
_F32 = jnp.float32

GCH = 128
GCHUNKS = E // GCH
GTRIPS = (GCHUNKS + NS - 1) // NS

SCH = 80
SCHUNKS = E // SCH
STRIPS = SCHUNKS // NS

RPT = 624
RPT_LAST = N - (NS - 1) * RPT


def _sc_gather_body(xw, src, xg_out, src_v, rows, sem_i0, sem_i1, sem_g0,
                    sem_g1):
    c = lax.axis_index("c")
    s = lax.axis_index("s")
    sem_i = (sem_i0, sem_i1)
    sem_g = (sem_g0, sem_g1)

    def _valid(u):
        return (s + NS * u) < GCHUNKS

    def _base(u):
        return c * E + (s + NS * u) * GCH

    def _issue_idx(u, b):
        @pl.when(_valid(u))
        def _():
            pltpu.async_copy(src.at[pl.ds(_base(u), GCH)], src_v.at[b],
                             sem_i[b])

    def _body(tt):
        for bi in range(2):
            t = 2 * tt + bi
            b = bi % 2
            nb = 1 - b

            @pl.when(_valid(t))
            def _gather_cur():
                pltpu.make_async_copy(src.at[pl.ds(0, GCH)], src_v.at[b],
                                      sem_i[b]).wait()
                pltpu.async_copy(xw.at[src_v.at[b]], rows.at[b], sem_g[b])

            @pl.when(jnp.logical_and(t >= 1, _valid(t - 1)))
            def _write_prev():
                pltpu.make_async_copy(xw.at[src_v.at[nb]], rows.at[nb],
                                      sem_g[nb]).wait()
                pltpu.sync_copy(rows.at[nb],
                                xg_out.at[pl.ds(_base(t - 1), GCH)])

            _issue_idx(t + 1, nb)

    _issue_idx(0, 0)
    pl.loop(0, (GTRIPS + 2) // 2)(_body)


@functools.cache
def _sc_gather_kernel():
    return pl.kernel(
        _sc_gather_body,
        out_type=jax.ShapeDtypeStruct((2 * E, H), _F32),
        mesh=plsc.VectorSubcoreMesh(core_axis_name="c", subcore_axis_name="s",
                                    num_cores=2, num_subcores=NS),
        scratch_types=[
            pltpu.VMEM((2, GCH), jnp.int32),
            pltpu.VMEM((2, GCH, H), _F32),
            pltpu.SemaphoreType.DMA,
            pltpu.SemaphoreType.DMA,
            pltpu.SemaphoreType.DMA,
            pltpu.SemaphoreType.DMA,
        ],
    )


def _sc_gather(xw, src):
    return _sc_gather_kernel()(xw, src)


def _sc_scatter_body(m, dst, agg_out, dst_v, m_in, agg_sp, sem_i0, sem_i1,
                     sem_m0, sem_m1):
    c = lax.axis_index("c")
    s = lax.axis_index("s")
    sem_i = (sem_i0, sem_i1)
    sem_m = (sem_m0, sem_m1)

    def _zrow(i, carry):
        for v in range(H // 16):
            m_in[0, i, pl.ds(v * 16, 16)] = jnp.zeros((16,), _F32)
        return carry
    lax.fori_loop(0, SCH, _zrow, 0)
    row0 = s * RPT

    @pl.when(s < NS - 1)
    def _zero_mid():
        off = 0
        while off < RPT:
            sz = min(SCH, RPT - off)
            pltpu.sync_copy(m_in.at[0].at[pl.ds(0, sz)],
                            agg_sp.at[pl.ds(row0 + off, sz)])
            off += sz

    @pl.when(s == NS - 1)
    def _zero_last():
        off = 0
        while off < RPT_LAST:
            sz = min(SCH, RPT_LAST - off)
            pltpu.sync_copy(m_in.at[0].at[pl.ds(0, sz)],
                            agg_sp.at[pl.ds(row0 + off, sz)])
            off += sz

    plsc.subcore_barrier()

    def _base(u):
        return c * E + (s + NS * u) * SCH

    def _issue(u, b):
        pltpu.async_copy(dst.at[pl.ds(_base(u), SCH)], dst_v.at[b], sem_i[b])
        pltpu.async_copy(m.at[pl.ds(_base(u), SCH)], m_in.at[b], sem_m[b])

    def _body(tt):
        for bi in range(2):
            t = 2 * tt + bi
            b = bi % 2
            nb = 1 - b

            @pl.when(t + 1 < STRIPS)
            def _prefetch():
                _issue(t + 1, nb)

            pltpu.make_async_copy(dst.at[pl.ds(0, SCH)], dst_v.at[b],
                                  sem_i[b]).wait()
            pltpu.make_async_copy(m.at[pl.ds(0, SCH)], m_in.at[b],
                                  sem_m[b]).wait()
            pltpu.sync_copy(m_in.at[b], agg_sp.at[dst_v.at[b]], add=True)

    _issue(0, 0)
    pl.loop(0, STRIPS // 2)(_body)

    plsc.subcore_barrier()

    @pl.when(s < NS - 1)
    def _out_mid():
        pltpu.sync_copy(agg_sp.at[pl.ds(s * RPT, RPT)],
                        agg_out.at[pl.ds(c * N + s * RPT, RPT)])

    @pl.when(s == NS - 1)
    def _out_last():
        pltpu.sync_copy(agg_sp.at[pl.ds(s * RPT, RPT_LAST)],
                        agg_out.at[pl.ds(c * N + s * RPT, RPT_LAST)])


@functools.cache
def _sc_scatter_kernel():
    return pl.kernel(
        _sc_scatter_body,
        out_type=jax.ShapeDtypeStruct((2 * N, H), _F32),
        mesh=plsc.VectorSubcoreMesh(core_axis_name="c", subcore_axis_name="s",
                                    num_cores=2, num_subcores=NS),
        scratch_types=[
            pltpu.VMEM((2, SCH), jnp.int32),
            pltpu.VMEM((2, SCH, H), _F32),
            pltpu.VMEM_SHARED((N, H), _F32),
            pltpu.SemaphoreType.DMA,
            pltpu.SemaphoreType.DMA,
            pltpu.SemaphoreType.DMA,
            pltpu.SemaphoreType.DMA,
        ],
    )


def _sc_scatter(m, dst):
    return _sc_scatter_kernel()(m, dst)


_NBLK = 2000
_EBLK = 4000
_NHALF = N // _NBLK
_EHALF = E // _EBLK


def _dot(a, b):
    return jnp.dot(a, b, preferred_element_type=_F32,
                   precision=lax.Precision.DEFAULT)


def _prep_x_body(nfa, nfb, Wn, bn, x0):
    i = pl.program_id(0)

    @pl.when(i < _NHALF)
    def _():
        x0[...] = _dot(nfa[...], Wn[...]) + bn[...]

    @pl.when(i >= _NHALF)
    def _():
        x0[...] = _dot(nfb[...], Wn[...]) + bn[...]


def _prep_x(nfa, nfb, Wn, bn):
    return pl.pallas_call(
        _prep_x_body,
        grid=(2 * N // _NBLK,),
        in_specs=[
            pl.BlockSpec((_NBLK, H), lambda i: (jnp.minimum(i, _NHALF - 1),
                                                0)),
            pl.BlockSpec((_NBLK, H), lambda i: (jnp.maximum(i - _NHALF, 0),
                                                0)),
            pl.BlockSpec((H, H), lambda i: (0, 0)),
            pl.BlockSpec((1, H), lambda i: (0, 0)),
        ],
        out_specs=pl.BlockSpec((_NBLK, H), lambda i: (i, 0)),
        out_shape=jax.ShapeDtypeStruct((2 * N, H), _F32),
    )(nfa, nfb, Wn, bn)


def _prep_e_body(efa, efb, We, be, e):
    i = pl.program_id(0)

    @pl.when(i < _EHALF)
    def _():
        e[...] = _dot(efa[...], We[...]) + be[...]

    @pl.when(i >= _EHALF)
    def _():
        e[...] = _dot(efb[...], We[...]) + be[...]


def _prep_e(efa, efb, We, be):
    return pl.pallas_call(
        _prep_e_body,
        grid=(2 * E // _EBLK,),
        in_specs=[
            pl.BlockSpec((_EBLK, DB), lambda i: (jnp.minimum(i, _EHALF - 1),
                                                 0)),
            pl.BlockSpec((_EBLK, DB), lambda i: (jnp.maximum(i - _EHALF, 0),
                                                 0)),
            pl.BlockSpec((DB, H), lambda i: (0, 0)),
            pl.BlockSpec((1, H), lambda i: (0, 0)),
        ],
        out_specs=pl.BlockSpec((_EBLK, H), lambda i: (i, 0)),
        out_shape=jax.ShapeDtypeStruct((2 * E, H), _F32),
    )(efa, efb, We, be)


def _msg_body(xg, e, Wm, bm, mo):
    mi = jnp.concatenate([xg[...], e[...]], axis=1)
    mo[...] = jnp.maximum(_dot(mi, Wm[...]) + bm[...], 0.0)


def _msg(xg, e, Wm, bm):
    return pl.pallas_call(
        _msg_body,
        grid=(2 * E // _EBLK,),
        in_specs=[
            pl.BlockSpec((_EBLK, H), lambda i: (i, 0)),
            pl.BlockSpec((_EBLK, H), lambda i: (i, 0)),
            pl.BlockSpec((2 * H, H), lambda i: (0, 0)),
            pl.BlockSpec((1, H), lambda i: (0, 0)),
        ],
        out_specs=pl.BlockSpec((_EBLK, H), lambda i: (i, 0)),
        out_shape=jax.ShapeDtypeStruct((2 * E, H), _F32),
    )(xg, e, Wm, bm)


def _ln_update(x, agg, Wu, bu, lg, lb):
    xn = _dot(jnp.concatenate([x, agg], axis=1), Wu) + bu
    t = x + xn
    mu = jnp.mean(t, axis=1, keepdims=True)
    var = jnp.mean((t - mu) ** 2, axis=1, keepdims=True)
    return (t - mu) / jnp.sqrt(var + 1e-5) * lg + lb


def _upd_body(x, agg, Wu, bu, lg, lb, xo):
    xo[...] = _ln_update(x[...], agg[...], Wu[...], bu[...], lg[...], lb[...])


def _upd_mm_body(x, agg, Wu, bu, to):
    to[...] = x[...] + _dot(jnp.concatenate([x[...], agg[...]], axis=1),
                            Wu[...]) + bu[...]


def _upd(x, agg, Wu, bu, lg, lb):
    nspec = pl.BlockSpec((_NBLK, H), lambda i: (i, 0))
    t = pl.pallas_call(
        _upd_mm_body,
        grid=(2 * N // _NBLK,),
        in_specs=[nspec, nspec,
                  pl.BlockSpec((2 * H, H), lambda i: (0, 0)),
                  pl.BlockSpec((1, H), lambda i: (0, 0))],
        out_specs=nspec,
        out_shape=jax.ShapeDtypeStruct((2 * N, H), _F32),
    )(x, agg, Wu, bu)
    mu = jnp.mean(t, axis=-1, keepdims=True)
    var = jnp.var(t, axis=-1, keepdims=True)
    return (t - mu) / jnp.sqrt(var + 1e-5) * lg + lb


def _upd_pool_body(x, agg, Wu, bu, lg, lb, batch, pooled, counts):
    i = pl.program_id(0)
    xn = _ln_update(x[...], agg[...], Wu[...], bu[...], lg[...], lb[...])
    gid = lax.broadcasted_iota(jnp.int32, (_NBLK, 2 * G), 1)
    oh = (batch[...] == gid).astype(_F32)
    p = lax.dot_general(oh, xn, (((0,), (0,)), ((), ())),
                        preferred_element_type=_F32,
                        precision=lax.Precision.HIGHEST)
    cnt = lax.dot_general(oh, jnp.ones((_NBLK, 1), _F32),
                          (((0,), (0,)), ((), ())),
                          preferred_element_type=_F32,
                          precision=lax.Precision.HIGHEST)

    @pl.when(i == 0)
    def _():
        pooled[...] = jnp.zeros_like(pooled)
        counts[...] = jnp.zeros_like(counts)

    pooled[...] += p
    counts[...] += cnt


def _upd_pool(x, agg, Wu, bu, lg, lb, batch):
    nspec = pl.BlockSpec((_NBLK, H), lambda i: (i, 0))
    return pl.pallas_call(
        _upd_pool_body,
        grid=(2 * N // _NBLK,),
        in_specs=[nspec, nspec,
                  pl.BlockSpec((2 * H, H), lambda i: (0, 0)),
                  pl.BlockSpec((1, H), lambda i: (0, 0)),
                  pl.BlockSpec((1, H), lambda i: (0, 0)),
                  pl.BlockSpec((1, H), lambda i: (0, 0)),
                  pl.BlockSpec((_NBLK, 1), lambda i: (i, 0))],
        out_specs=[pl.BlockSpec((2 * G, H), lambda i: (0, 0)),
                   pl.BlockSpec((2 * G, 1), lambda i: (0, 0))],
        out_shape=[jax.ShapeDtypeStruct((2 * G, H), _F32),
                   jax.ShapeDtypeStruct((2 * G, 1), _F32)],
    )(x, agg, Wu, bu, lg, lb, batch)


def _head_body(pooled, counts, pW1, pb1, pW2, pb2, L, hW1, hb1, hW2, hb2,
               hW3, hb3, out):
    mean = pooled[...] / jnp.maximum(counts[...], 1.0)
    h = _dot(jnp.maximum(_dot(mean, pW1[...]) + pb1[...], 0.0), pW2[...]) \
        + pb2[...]
    ha = h[:G]
    hb = h[G:]
    Lm = L[...]
    M = lax.dot_general(Lm, Lm, (((1,), (1,)), ((), ())),
                        preferred_element_type=_F32,
                        precision=lax.Precision.DEFAULT)
    delta = hb - ha
    d_sq = jnp.sum(delta * _dot(delta, M), axis=1, keepdims=True)
    d_m = jnp.sqrt(d_sq + 1e-8)
    feats = jnp.concatenate([d_m, delta, ha + hb], axis=1)
    z = jnp.maximum(_dot(feats, hW1[...]) + hb1[...], 0.0)
    z = jnp.maximum(_dot(z, hW2[...]) + hb2[...], 0.0)
    out[...] = _dot(z, hW3[...]) + hb3[...]


def _head(pooled, counts, p):
    diag = jax.nn.softplus(p['L_diag']) + 0.01
    L = jnp.diag(diag).at[_TRIL_R, _TRIL_C].set(p['L_lower'])
    args = (pooled, counts, p['pW1'], p['pb1'].reshape(1, H), p['pW2'],
            p['pb2'].reshape(1, H), L, p['hW1'], p['hb1'].reshape(1, H),
            p['hW2'], p['hb2'].reshape(1, H // 2), p['hW3'],
            p['hb3'].reshape(1, 1))
    return pl.pallas_call(
        _head_body,
        out_shape=jax.ShapeDtypeStruct((G, 1), _F32),
    )(*args)


_TRIL_R, _TRIL_C = np.tril_indices(H, -1)


def kernel(node_feats_a, edge_feats_a, edge_index_a, batch_a, node_feats_b,
           edge_feats_b, edge_index_b, batch_b, params):
    p = params
    src = jnp.concatenate([edge_index_a[0], edge_index_b[0] + N], axis=0)
    dst = jnp.concatenate([edge_index_a[1], edge_index_b[1]], axis=0)
    batch = jnp.concatenate([batch_a, batch_b + G], axis=0).reshape(2 * N, 1)

    x = _prep_x(node_feats_a, node_feats_b, p['Wn'], p['bn'].reshape(1, H))
    e = _prep_e(edge_feats_a, edge_feats_b, p['We'], p['be'].reshape(1, H))

    for l in range(NL):
        xg = _sc_gather(x, src)
        m = _msg(xg, e, p['Wm%d' % l], p['bm%d' % l].reshape(1, H))
        agg = _sc_scatter(m, dst)
        if l + 1 < NL:
            x = _upd(x, agg, p['Wu%d' % l], p['bu%d' % l].reshape(1, H),
                     p['lg%d' % l].reshape(1, H), p['lb%d' % l].reshape(1, H))
        else:
            pooled, counts = _upd_pool(x, agg, p['Wu%d' % l],
                                       p['bu%d' % l].reshape(1, H),
                                       p['lg%d' % l].reshape(1, H),
                                       p['lb%d' % l].reshape(1, H), batch)

    out = _head(pooled, counts, p)
    return out[:, 0]
